# baseline (speedup 1.0000x reference)
import functools

import jax
import jax.numpy as jnp
from jax.experimental import pallas as pl
from jax.experimental.pallas import tpu as pltpu

_HIDDEN = 128
_SLOTS = 512
_VOCAB = 128
_BB = 16
_NSPLIT = 4


def _attn_body(scal_ref, enc_ref, query_ref, wq_ref, bq_ref, wk_ref, bk_ref,
               wout_ref, bout_ref, out_ref, buf_ref, sem_ref):
    L = scal_ref[0]
    w0 = scal_ref[1]
    nblk = pl.num_programs(0)
    i = pl.program_id(0)
    scale = 1.0 / (_HIDDEN ** 0.5)

    def window_copies(blk, slot):
        sub = _BB // _NSPLIT
        return [
            pltpu.make_async_copy(
                enc_ref.at[pl.ds(blk * _BB + j * sub, sub),
                           pl.ds(w0, _SLOTS), :],
                buf_ref.at[slot, pl.ds(j * sub, sub)],
                sem_ref.at[slot, j],
            )
            for j in range(_NSPLIT)
        ]

    @pl.when(i == 0)
    def _():
        for c in window_copies(0, 0):
            c.start()

    @pl.when(i + 1 < nblk)
    def _():
        for c in window_copies(i + 1, (i + 1) % 2):
            c.start()

    query = query_ref[...]
    q = jax.lax.dot_general(query, wq_ref[...], (((1,), (1,)), ((), ())),
                            preferred_element_type=jnp.float32) + bq_ref[...]
    qk = jax.lax.dot_general(q, wk_ref[...], (((1,), (0,)), ((), ())),
                             preferred_element_type=jnp.float32)
    qb = jnp.sum(q * bk_ref[...], axis=1, keepdims=True)

    for c in window_copies(i, i % 2):
        c.wait()
    win = buf_ref[i % 2]
    row = jax.lax.broadcasted_iota(jnp.int32, (1, _SLOTS), 1)
    written = (w0 + row) < L

    dots = jnp.concatenate([
        jax.lax.dot_general(qk[b:b + 1], win[b], (((1,), (1,)), ((), ())),
                            preferred_element_type=jnp.float32)
        for b in range(_BB)
    ], axis=0)
    scores = (jnp.where(written, dots, 0.0) + qb) * scale
    m = jnp.max(scores, axis=1, keepdims=True)
    p = jnp.exp(scores - m)
    denom = jnp.sum(p, axis=1, keepdims=True)
    w = jnp.where(written, p, 0.0) / denom
    retrieved = jnp.concatenate([
        jax.lax.dot_general(w[b:b + 1], win[b], (((1,), (0,)), ((), ())),
                            preferred_element_type=jnp.float32)
        for b in range(_BB)
    ], axis=0)

    x = retrieved + query
    out_ref[...] = jax.lax.dot_general(
        x, wout_ref[...], (((1,), (1,)), ((), ())),
        preferred_element_type=jnp.float32) + bout_ref[...]


@functools.partial(jax.jit, static_argnums=())
def kernel(enc_hidden, query_hidden, Wq, bq, Wk, bk, Wout, bout, num_pairs):
    B, T, H = enc_hidden.shape
    L = jnp.minimum(jnp.asarray(num_pairs, jnp.int32) * 2, T - 3)
    w0 = jnp.maximum(L - _SLOTS, 0)
    scal = jnp.stack([L, w0]).astype(jnp.int32)

    grid = (B // _BB,)
    out = pl.pallas_call(
        _attn_body,
        grid=grid,
        in_specs=[
            pl.BlockSpec(memory_space=pltpu.SMEM),
            pl.BlockSpec(memory_space=pl.ANY),
            pl.BlockSpec((_BB, H), lambda i: (i, 0)),
            pl.BlockSpec((H, H), lambda i: (0, 0)),
            pl.BlockSpec((1, H), lambda i: (0, 0)),
            pl.BlockSpec((H, H), lambda i: (0, 0)),
            pl.BlockSpec((1, H), lambda i: (0, 0)),
            pl.BlockSpec((_VOCAB, H), lambda i: (0, 0)),
            pl.BlockSpec((1, _VOCAB), lambda i: (0, 0)),
        ],
        out_specs=pl.BlockSpec((_BB, _VOCAB), lambda i: (i, 0)),
        out_shape=jax.ShapeDtypeStruct((B, _VOCAB), jnp.float32),
        scratch_shapes=[
            pltpu.VMEM((2, _BB, _SLOTS, H), jnp.float32),
            pltpu.SemaphoreType.DMA((2, _NSPLIT)),
        ],
    )(scal, enc_hidden, query_hidden, Wq, bq.reshape(1, H), Wk,
      bk.reshape(1, H), Wout, bout.reshape(1, _VOCAB))
    return out


# speedup vs baseline: 1.1361x; 1.1361x over previous
import functools

import jax
import jax.numpy as jnp
from jax.experimental import pallas as pl
from jax.experimental.pallas import tpu as pltpu

_HIDDEN = 128
_SLOTS = 512
_VOCAB = 128
_BB = 16
_NSPLIT = 4
_NBUF = 3


def _attn_body(scal_ref, enc_ref, query_ref, wq_ref, bq_ref, wk_ref, bk_ref,
               wout_ref, bout_ref, out_ref, buf_ref, sem_ref):
    L = scal_ref[0]
    w0 = scal_ref[1]
    nblk = pl.num_programs(0)
    i = pl.program_id(0)
    scale = 1.0 / (_HIDDEN ** 0.5)

    def window_copies(blk, slot):
        sub = _BB // _NSPLIT
        return [
            pltpu.make_async_copy(
                enc_ref.at[pl.ds(blk * _BB + j * sub, sub),
                           pl.ds(w0, _SLOTS), :],
                buf_ref.at[slot, pl.ds(j * sub, sub)],
                sem_ref.at[slot, j],
            )
            for j in range(_NSPLIT)
        ]

    @pl.when(i == 0)
    def _():
        for c in window_copies(0, 0):
            c.start()
        for c in window_copies(1, 1):
            c.start()

    @pl.when(i + 2 < nblk)
    def _():
        for c in window_copies(i + 2, (i + 2) % _NBUF):
            c.start()

    query = query_ref[...]
    q = jax.lax.dot_general(query, wq_ref[...], (((1,), (1,)), ((), ())),
                            preferred_element_type=jnp.float32) + bq_ref[...]
    qk = jax.lax.dot_general(q, wk_ref[...], (((1,), (0,)), ((), ())),
                             preferred_element_type=jnp.float32)
    qb = jnp.sum(q * bk_ref[...], axis=1, keepdims=True)

    for c in window_copies(i, i % _NBUF):
        c.wait()
    win = buf_ref[i % _NBUF]
    row = jax.lax.broadcasted_iota(jnp.int32, (1, _SLOTS), 1)
    written = (w0 + row) < L

    dots = jnp.concatenate([
        jax.lax.dot_general(qk[b:b + 1], win[b], (((1,), (1,)), ((), ())),
                            preferred_element_type=jnp.float32)
        for b in range(_BB)
    ], axis=0)
    scores = (jnp.where(written, dots, 0.0) + qb) * scale
    m = jnp.max(scores, axis=1, keepdims=True)
    p = jnp.exp(scores - m)
    denom = jnp.sum(p, axis=1, keepdims=True)
    w = jnp.where(written, p, 0.0) / denom
    retrieved = jnp.concatenate([
        jax.lax.dot_general(w[b:b + 1], win[b], (((1,), (0,)), ((), ())),
                            preferred_element_type=jnp.float32)
        for b in range(_BB)
    ], axis=0)

    x = retrieved + query
    out_ref[...] = jax.lax.dot_general(
        x, wout_ref[...], (((1,), (1,)), ((), ())),
        preferred_element_type=jnp.float32) + bout_ref[...]


@functools.partial(jax.jit, static_argnums=())
def kernel(enc_hidden, query_hidden, Wq, bq, Wk, bk, Wout, bout, num_pairs):
    B, T, H = enc_hidden.shape
    L = jnp.minimum(jnp.asarray(num_pairs, jnp.int32) * 2, T - 3)
    w0 = jnp.maximum(L - _SLOTS, 0)
    scal = jnp.stack([L, w0]).astype(jnp.int32)

    grid = (B // _BB,)
    out = pl.pallas_call(
        _attn_body,
        grid=grid,
        in_specs=[
            pl.BlockSpec(memory_space=pltpu.SMEM),
            pl.BlockSpec(memory_space=pl.ANY),
            pl.BlockSpec((_BB, H), lambda i: (i, 0)),
            pl.BlockSpec((H, H), lambda i: (0, 0)),
            pl.BlockSpec((1, H), lambda i: (0, 0)),
            pl.BlockSpec((H, H), lambda i: (0, 0)),
            pl.BlockSpec((1, H), lambda i: (0, 0)),
            pl.BlockSpec((_VOCAB, H), lambda i: (0, 0)),
            pl.BlockSpec((1, _VOCAB), lambda i: (0, 0)),
        ],
        out_specs=pl.BlockSpec((_BB, _VOCAB), lambda i: (i, 0)),
        out_shape=jax.ShapeDtypeStruct((B, _VOCAB), jnp.float32),
        scratch_shapes=[
            pltpu.VMEM((_NBUF, _BB, _SLOTS, H), jnp.float32),
            pltpu.SemaphoreType.DMA((_NBUF, _NSPLIT)),
        ],
    )(scal, enc_hidden, query_hidden, Wq, bq.reshape(1, H), Wk,
      bk.reshape(1, H), Wout, bout.reshape(1, _VOCAB))
    return out
